# Initial kernel scaffold; baseline (speedup 1.0000x reference)
import jax
import jax.numpy as jnp
from jax.experimental import pallas as pl


def kernel(inputs, W_shared, b_shared, W_cls, b_cls, W_reg, b_reg):
    raise NotImplementedError("write your pallas kernel here")


# speedup vs baseline: 1.3398x; 1.3398x over previous
import functools

import jax
import jax.numpy as jnp
from jax.experimental import pallas as pl

_TR = 16


def _rpn_body(x_ref, w9_ref, bsh_ref, wh_ref, bh_ref,
              cls_ref, probs_ref, reg_ref, *, tr, h, w, c, n):
    i = pl.program_id(1)
    row0 = i * tr
    m = tr * w
    cols = []
    for dy in range(3):
        xs = x_ref[0, pl.ds(row0 + dy, tr), :, :]
        for dx in range(3):
            cols.append(xs[:, dx:dx + w, :].reshape(m, c))
    patch = jnp.concatenate(cols, axis=1)
    acc = jnp.dot(patch, w9_ref[...], preferred_element_type=jnp.float32)
    shared = jnp.maximum(acc + bsh_ref[...], 0.0).astype(jnp.bfloat16)
    head = (jnp.dot(shared, wh_ref[...], preferred_element_type=jnp.float32)
            + bh_ref[...])
    cls = head[:, :6]
    reg = head[:, 6:18]
    rot_l = jnp.concatenate([cls[:, 1:], cls[:, :1]], axis=1)
    rot_r = jnp.concatenate([cls[:, 5:], cls[:, :5]], axis=1)
    lane = jax.lax.broadcasted_iota(jnp.int32, cls.shape, 1)
    swapped = jnp.where(lane % 2 == 0, rot_l, rot_r)
    probs = jax.nn.sigmoid(cls - swapped)
    cls_ref[0] = cls
    probs_ref[0] = probs
    reg_ref[0] = reg


@jax.jit
def kernel(inputs, W_shared, b_shared, W_cls, b_cls, W_reg, b_reg):
    B, H, W, C = inputs.shape
    N = W_shared.shape[-1]
    A = W_cls.shape[-1] // 2

    x = jnp.pad(inputs, ((0, 0), (1, 1), (1, 1), (0, 0))).astype(jnp.bfloat16)
    w9 = W_shared.reshape(9 * C, N).astype(jnp.bfloat16)
    wh = jnp.concatenate([W_cls.reshape(N, 2 * A),
                          W_reg.reshape(N, 4 * A)], axis=1).astype(jnp.bfloat16)
    bsh = b_shared.reshape(1, N)
    bh = jnp.concatenate([b_cls, b_reg]).reshape(1, 6 * A)

    tr = _TR
    nt = H // tr
    m = tr * W
    body = functools.partial(_rpn_body, tr=tr, h=H, w=W, c=C, n=N)
    cls, probs, reg = pl.pallas_call(
        body,
        grid=(B, nt),
        in_specs=[
            pl.BlockSpec((1, H + 2, W + 2, C), lambda b, i: (b, 0, 0, 0)),
            pl.BlockSpec((9 * C, N), lambda b, i: (0, 0)),
            pl.BlockSpec((1, N), lambda b, i: (0, 0)),
            pl.BlockSpec((N, 6 * A), lambda b, i: (0, 0)),
            pl.BlockSpec((1, 6 * A), lambda b, i: (0, 0)),
        ],
        out_specs=[
            pl.BlockSpec((1, m, 2 * A), lambda b, i: (b, i, 0)),
            pl.BlockSpec((1, m, 2 * A), lambda b, i: (b, i, 0)),
            pl.BlockSpec((1, m, 4 * A), lambda b, i: (b, i, 0)),
        ],
        out_shape=[
            jax.ShapeDtypeStruct((B, H * W, 2 * A), jnp.float32),
            jax.ShapeDtypeStruct((B, H * W, 2 * A), jnp.float32),
            jax.ShapeDtypeStruct((B, H * W, 4 * A), jnp.float32),
        ],
    )(x, w9, bsh, wh, bh)

    rpn_class_logits = cls.reshape(B, H * W * A, 2)
    rpn_probs = probs.reshape(B, H * W * A, 2)
    rpn_deltas = reg.reshape(B, H * W * A, 4)
    return (rpn_class_logits, rpn_probs, rpn_deltas)


# speedup vs baseline: 1.3763x; 1.0272x over previous
import functools

import jax
import jax.numpy as jnp
from jax.experimental import pallas as pl
from jax.experimental.pallas import tpu as pltpu

_TR = 16


def _rpn_body(x_ref, w9_ref, bsh_ref, wh_ref, bh_ref,
              cls_ref, probs_ref, reg_ref, xs, w_bf, wh_bf,
              *, tr, h, w, c, n, nt):
    b = pl.program_id(0)
    i = pl.program_id(1)
    row0 = i * tr
    m = tr * w

    @pl.when((b == 0) & (i == 0))
    def _init():
        w_bf[...] = w9_ref[...].astype(jnp.bfloat16)
        wh_bf[...] = wh_ref[...].astype(jnp.bfloat16)
        xs[:, 0:1, :] = jnp.zeros((tr + 2, 1, c), jnp.bfloat16)
        xs[:, w + 1:w + 2, :] = jnp.zeros((tr + 2, 1, c), jnp.bfloat16)

    @pl.when(i == 0)
    def _top():
        xs[0:1, 1:w + 1, :] = jnp.zeros((1, w, c), jnp.bfloat16)
        xs[1:tr + 2, 1:w + 1, :] = x_ref[0, 0:tr + 1, :, :].astype(jnp.bfloat16)

    @pl.when((i > 0) & (i < nt - 1))
    def _mid():
        xs[0:tr + 2, 1:w + 1, :] = (
            x_ref[0, pl.ds(row0 - 1, tr + 2), :, :].astype(jnp.bfloat16))

    @pl.when(i == nt - 1)
    def _bot():
        xs[tr + 1:tr + 2, 1:w + 1, :] = jnp.zeros((1, w, c), jnp.bfloat16)
        xs[0:tr + 1, 1:w + 1, :] = (
            x_ref[0, pl.ds(row0 - 1, tr + 1), :, :].astype(jnp.bfloat16))

    cols = []
    for dy in range(3):
        for dx in range(3):
            cols.append(xs[dy:dy + tr, dx:dx + w, :].reshape(m, c))
    patch = jnp.concatenate(cols, axis=1)
    acc = jnp.dot(patch, w_bf[...], preferred_element_type=jnp.float32)
    shared = jnp.maximum(acc + bsh_ref[...], 0.0).astype(jnp.bfloat16)
    head = (jnp.dot(shared, wh_bf[...], preferred_element_type=jnp.float32)
            + bh_ref[...])
    cls = head[:, :6]
    reg = head[:, 6:18]
    rot_l = jnp.concatenate([cls[:, 1:], cls[:, :1]], axis=1)
    rot_r = jnp.concatenate([cls[:, 5:], cls[:, :5]], axis=1)
    lane = jax.lax.broadcasted_iota(jnp.int32, cls.shape, 1)
    swapped = jnp.where(lane % 2 == 0, rot_l, rot_r)
    probs = jax.nn.sigmoid(cls - swapped)
    comb = jnp.concatenate([cls, probs, reg], axis=1)
    comb_t = comb.T
    cls_ref[0] = comb_t[0:6].reshape(3, 2, m)
    probs_ref[0] = comb_t[6:12].reshape(3, 2, m)
    reg_ref[0] = comb_t[12:24].reshape(3, 4, m)


@jax.jit
def kernel(inputs, W_shared, b_shared, W_cls, b_cls, W_reg, b_reg):
    B, H, W, C = inputs.shape
    N = W_shared.shape[-1]
    A = W_cls.shape[-1] // 2
    HW = H * W

    w9 = W_shared.reshape(9 * C, N)
    wh = jnp.concatenate([W_cls.reshape(N, 2 * A),
                          W_reg.reshape(N, 4 * A)], axis=1)
    bsh = b_shared.reshape(1, N)
    bh = jnp.concatenate([b_cls, b_reg]).reshape(1, 6 * A)

    tr = _TR
    nt = H // tr
    m = tr * W
    body = functools.partial(_rpn_body, tr=tr, h=H, w=W, c=C, n=N, nt=nt)
    cls_t, probs_t, reg_t = pl.pallas_call(
        body,
        grid=(B, nt),
        in_specs=[
            pl.BlockSpec((1, H, W, C), lambda b, i: (b, 0, 0, 0)),
            pl.BlockSpec((9 * C, N), lambda b, i: (0, 0)),
            pl.BlockSpec((1, N), lambda b, i: (0, 0)),
            pl.BlockSpec((N, 6 * A), lambda b, i: (0, 0)),
            pl.BlockSpec((1, 6 * A), lambda b, i: (0, 0)),
        ],
        out_specs=[
            pl.BlockSpec((1, A, 2, m), lambda b, i: (b, 0, 0, i)),
            pl.BlockSpec((1, A, 2, m), lambda b, i: (b, 0, 0, i)),
            pl.BlockSpec((1, A, 4, m), lambda b, i: (b, 0, 0, i)),
        ],
        out_shape=[
            jax.ShapeDtypeStruct((B, A, 2, HW), jnp.float32),
            jax.ShapeDtypeStruct((B, A, 2, HW), jnp.float32),
            jax.ShapeDtypeStruct((B, A, 4, HW), jnp.float32),
        ],
        scratch_shapes=[
            pltpu.VMEM((tr + 2, W + 2, C), jnp.bfloat16),
            pltpu.VMEM((9 * C, N), jnp.bfloat16),
            pltpu.VMEM((N, 6 * A), jnp.bfloat16),
        ],
    )(inputs, w9, bsh, wh, bh)

    rpn_class_logits = cls_t.transpose(0, 3, 1, 2).reshape(B, HW * A, 2)
    rpn_probs = probs_t.transpose(0, 3, 1, 2).reshape(B, HW * A, 2)
    rpn_deltas = reg_t.transpose(0, 3, 1, 2).reshape(B, HW * A, 4)
    return (rpn_class_logits, rpn_probs, rpn_deltas)
